# Initial kernel scaffold; baseline (speedup 1.0000x reference)
import jax
import jax.numpy as jnp
from jax.experimental import pallas as pl


def kernel(list_neigh, Imagetype_map, atom_type, ImageDR, num_neigh, nghost, c_param, W0, b0, W1, b1, W2, b2):
    raise NotImplementedError("write your pallas kernel here")


# speedup vs baseline: 1.4387x; 1.4387x over previous
import functools

import jax
import jax.numpy as jnp
import numpy as np
from jax.experimental import pallas as pl

RC = 6.0
RMIN = 0.5
BETA = 12
M1 = 8
M2 = 8
HID = 64
NT = 2
MN = 64
LW = NT * MN * 4

_NA = 400


def _body(dr_ref, nn_ref, tm_ref, C_ref, W0_ref, b0_ref, W1_ref, b1_ref,
          W2_ref, b2_ref, Rrep_ref, ei_ref, etot_ref):
    mm = functools.partial(
        jax.lax.dot_general,
        dimension_numbers=(((1,), (0,)), ((), ())),
        precision=jax.lax.Precision.HIGHEST,
        preferred_element_type=jnp.float32,
    )
    d = dr_ref[...]
    na = d.shape[0]
    lane = jax.lax.broadcasted_iota(jnp.int32, (na, LW), 1)
    cch = lane & 3
    j = (lane >> 2) & (MN - 1)
    k = lane >> 8
    nnk = jnp.where(k == 0, nn_ref[:, 0:1], nn_ref[:, 1:2])
    valid = (cch == 0) & (j < nnk)
    r = d * RC
    s = jnp.clip((r - RMIN) / (RC - RMIN), 0.0, 1.0)
    fc = 1.0 - 3.0 * s * s + 2.0 * s * s * s
    fc = jnp.where(valid, fc, 0.0)
    x = jnp.clip(2.0 * r / RC - 1.0, -1.0, 1.0)

    tm2 = fc
    tm1 = fc * x
    g = mm(tm2, C_ref[0]) + mm(tm1, C_ref[1])
    for a in range(2, BETA):
        tc = 2.0 * x * tm1 - tm2
        g = g + mm(tc, C_ref[a])
        tm2, tm1 = tm1, tc

    is0 = tm_ref[...] == 0
    gsel = jnp.where(is0, g[:, :M1], g[:, M1:])
    tile = jnp.concatenate([gsel] * M2, axis=1)
    rep = mm(gsel, Rrep_ref[...])
    feat = rep * tile

    eis = []
    for t in range(NT):
        h = jnp.tanh(mm(feat, W0_ref[t]) + b0_ref[t])
        h = jnp.tanh(mm(h, W1_ref[t]) + b1_ref[t])
        eis.append(mm(h, W2_ref[t]) + b2_ref[t])
    ei = jnp.where(is0, eis[0], eis[1])
    ei_ref[...] = ei

    prev = jnp.where(pl.program_id(0) == 0, 0.0, etot_ref[0, 0])
    etot_ref[...] = (prev + jnp.sum(ei))[None, None]


def kernel(list_neigh, Imagetype_map, atom_type, ImageDR, num_neigh, nghost,
           c_param, W0, b0, W1, b1, W2, b2):
    B, N, _, _ = list_neigh.shape
    dr = ImageDR.reshape(N, LW)
    nn = num_neigh.reshape(N, NT).astype(jnp.int32)
    tm = Imagetype_map.reshape(N, 1).astype(jnp.int32)

    l = np.arange(LW)
    kk = jnp.asarray(l // (MN * 4))
    ch0 = jnp.asarray((l % 4 == 0).astype(np.float32))
    Cl = c_param[:, kk, :, :]
    C = (Cl * ch0[None, :, None, None] / MN).transpose(3, 1, 0, 2).reshape(BETA, LW, NT * M1)
    Rrep = jnp.asarray(np.kron(np.eye(M1, dtype=np.float32),
                               np.ones((1, M2), np.float32)))
    b0r = b0.reshape(NT, 1, HID)
    b1r = b1.reshape(NT, 1, HID)
    b2r = b2.reshape(NT, 1, 1)

    grid = N // _NA
    full = lambda *shape: pl.BlockSpec(shape, lambda i: (0,) * len(shape))
    ei, etot = pl.pallas_call(
        _body,
        grid=(grid,),
        in_specs=[
            pl.BlockSpec((_NA, LW), lambda i: (i, 0)),
            pl.BlockSpec((_NA, NT), lambda i: (i, 0)),
            pl.BlockSpec((_NA, 1), lambda i: (i, 0)),
            full(BETA, LW, NT * M1),
            full(NT, M1 * M2, HID),
            full(NT, 1, HID),
            full(NT, HID, HID),
            full(NT, 1, HID),
            full(NT, HID, 1),
            full(NT, 1, 1),
            full(M1, M1 * M2),
        ],
        out_specs=[
            pl.BlockSpec((_NA, 1), lambda i: (i, 0)),
            pl.BlockSpec((1, 1), lambda i: (0, 0)),
        ],
        out_shape=[
            jax.ShapeDtypeStruct((N, 1), jnp.float32),
            jax.ShapeDtypeStruct((1, 1), jnp.float32),
        ],
    )(dr, nn, tm, C, W0, b0r, W1, b1r, W2, b2r, Rrep)
    return etot.reshape(B, 1), ei.reshape(B, N)


# speedup vs baseline: 2.0117x; 1.3983x over previous
import functools

import jax
import jax.numpy as jnp
import numpy as np
from jax.experimental import pallas as pl

RC = 6.0
RMIN = 0.5
BETA = 12
M1 = 8
M2 = 8
HID = 64
NT = 2
MN = 64
LW = NT * MN * 4

_NA = 400


def _body(dr_ref, nn_ref, tm_ref, Sel_ref, C_ref, W0_ref, b0_ref, W1_ref,
          b1_ref, W2_ref, b2_ref, Rrep_ref, ei_ref, etot_ref):
    mm = functools.partial(
        jax.lax.dot_general,
        dimension_numbers=(((1,), (0,)), ((), ())),
        precision=jax.lax.Precision.HIGHEST,
        preferred_element_type=jnp.float32,
    )
    d = dr_ref[...]
    na = d.shape[0]
    r = mm(d, Sel_ref[...]) * RC
    lane = jax.lax.broadcasted_iota(jnp.int32, (na, NT * MN), 1)
    j = lane & (MN - 1)
    k = lane >> 6
    nnk = jnp.where(k == 0, nn_ref[:, 0:1], nn_ref[:, 1:2])
    s = jnp.clip((r - RMIN) / (RC - RMIN), 0.0, 1.0)
    fc = 1.0 - 3.0 * s * s + 2.0 * s * s * s
    fc = jnp.where(j < nnk, fc, 0.0)
    x = jnp.clip(2.0 * r / RC - 1.0, -1.0, 1.0)

    tm2 = fc
    tm1 = fc * x
    g = mm(tm2, C_ref[0]) + mm(tm1, C_ref[1])
    for a in range(2, BETA):
        tc = 2.0 * x * tm1 - tm2
        g = g + mm(tc, C_ref[a])
        tm2, tm1 = tm1, tc

    is0 = tm_ref[...] == 0
    gsel = jnp.where(is0, g[:, :M1], g[:, M1:])
    tile = jnp.concatenate([gsel] * M2, axis=1)
    rep = mm(gsel, Rrep_ref[...])
    feat = rep * tile

    eis = []
    for t in range(NT):
        h = jnp.tanh(mm(feat, W0_ref[t]) + b0_ref[t])
        h = jnp.tanh(mm(h, W1_ref[t]) + b1_ref[t])
        eis.append(mm(h, W2_ref[t]) + b2_ref[t])
    ei = jnp.where(is0, eis[0], eis[1])
    ei_ref[...] = ei

    prev = jnp.where(pl.program_id(0) == 0, 0.0, etot_ref[0, 0])
    etot_ref[...] = (prev + jnp.sum(ei))[None, None]


def kernel(list_neigh, Imagetype_map, atom_type, ImageDR, num_neigh, nghost,
           c_param, W0, b0, W1, b1, W2, b2):
    B, N, _, _ = list_neigh.shape
    dr = ImageDR.reshape(N, LW)
    nn = num_neigh.reshape(N, NT).astype(jnp.int32)
    tm = Imagetype_map.reshape(N, 1).astype(jnp.int32)

    Sel = jnp.asarray(np.eye(LW, dtype=np.float32)[0::4].T.copy())
    kk = jnp.asarray(np.arange(NT * MN) // MN)
    Cl = c_param[:, kk, :, :]
    C = (Cl / MN).transpose(3, 1, 0, 2).reshape(BETA, NT * MN, NT * M1)
    Rrep = jnp.asarray(np.kron(np.eye(M1, dtype=np.float32),
                               np.ones((1, M2), np.float32)))
    b0r = b0.reshape(NT, 1, HID)
    b1r = b1.reshape(NT, 1, HID)
    b2r = b2.reshape(NT, 1, 1)

    grid = N // _NA
    full = lambda *shape: pl.BlockSpec(shape, lambda i: (0,) * len(shape))
    ei, etot = pl.pallas_call(
        _body,
        grid=(grid,),
        in_specs=[
            pl.BlockSpec((_NA, LW), lambda i: (i, 0)),
            pl.BlockSpec((_NA, NT), lambda i: (i, 0)),
            pl.BlockSpec((_NA, 1), lambda i: (i, 0)),
            full(LW, NT * MN),
            full(BETA, NT * MN, NT * M1),
            full(NT, M1 * M2, HID),
            full(NT, 1, HID),
            full(NT, HID, HID),
            full(NT, 1, HID),
            full(NT, HID, 1),
            full(NT, 1, 1),
            full(M1, M1 * M2),
        ],
        out_specs=[
            pl.BlockSpec((_NA, 1), lambda i: (i, 0)),
            pl.BlockSpec((1, 1), lambda i: (0, 0)),
        ],
        out_shape=[
            jax.ShapeDtypeStruct((N, 1), jnp.float32),
            jax.ShapeDtypeStruct((1, 1), jnp.float32),
        ],
    )(dr, nn, tm, Sel, C, W0, b0r, W1, b1r, W2, b2r, Rrep)
    return etot.reshape(B, 1), ei.reshape(B, N)


# speedup vs baseline: 2.3102x; 1.1484x over previous
import functools

import jax
import jax.numpy as jnp
import numpy as np
from jax.experimental import pallas as pl

RC = 6.0
RMIN = 0.5
BETA = 12
M1 = 8
M2 = 8
HID = 64
NT = 2
MN = 64
LW = NT * MN * 4

_NA = 400


def _body(dr_ref, nn_ref, tm_ref, Sel_ref, C_ref, W0c_ref, b0c_ref,
          Wbd_ref, b1c_ref, W2s_ref, b2c_ref, Rrep_ref, ei_ref, etot_ref):
    mm = functools.partial(
        jax.lax.dot_general,
        dimension_numbers=(((1,), (0,)), ((), ())),
        precision=jax.lax.Precision.HIGHEST,
        preferred_element_type=jnp.float32,
    )
    d = dr_ref[...]
    na = d.shape[0]
    r = mm(d, Sel_ref[...]) * RC
    lane = jax.lax.broadcasted_iota(jnp.int32, (na, NT * MN), 1)
    j = lane & (MN - 1)
    k = lane >> 6
    nnk = jnp.where(k == 0, nn_ref[:, 0:1], nn_ref[:, 1:2])
    s = jnp.clip((r - RMIN) / (RC - RMIN), 0.0, 1.0)
    fc = 1.0 - 3.0 * s * s + 2.0 * s * s * s
    fc = jnp.where(j < nnk, fc, 0.0)
    x = jnp.clip(2.0 * r / RC - 1.0, -1.0, 1.0)

    x2 = 2.0 * x
    us = [fc, fc * x]
    for a in range(2, BETA):
        us.append(x2 * us[-1] - us[-2])
    g = mm(jnp.concatenate(us, axis=1), C_ref[...])

    is0 = tm_ref[...] == 0
    gsel = jnp.where(is0, g[:, :M1], g[:, M1:])
    tile = jnp.concatenate([gsel] * M2, axis=1)
    rep = mm(gsel, Rrep_ref[0])
    feat = rep * tile

    h = jnp.tanh(mm(feat, W0c_ref[0]) + b0c_ref[0])
    h = jnp.tanh(mm(h, Wbd_ref[0]) + b1c_ref[0])
    hlane = jax.lax.broadcasted_iota(jnp.int32, (na, NT * HID), 1)
    tmask = (hlane < HID) == is0
    ei = mm(jnp.where(tmask, h, 0.0), W2s_ref[0])
    ei = ei + jnp.where(is0, b2c_ref[0, 0, 0], b2c_ref[0, 0, 1])
    ei_ref[...] = ei

    prev = jnp.where(pl.program_id(0) == 0, 0.0, etot_ref[0, 0])
    etot_ref[...] = (prev + jnp.sum(ei))[None, None]


def kernel(list_neigh, Imagetype_map, atom_type, ImageDR, num_neigh, nghost,
           c_param, W0, b0, W1, b1, W2, b2):
    B, N, _, _ = list_neigh.shape
    dr = ImageDR.reshape(N, LW)
    nn = num_neigh.reshape(N, NT).astype(jnp.int32)
    tm = Imagetype_map.reshape(N, 1).astype(jnp.int32)

    Sel = jnp.asarray(np.eye(LW, dtype=np.float32)[0::4].T.copy())
    kk = jnp.asarray(np.arange(NT * MN) // MN)
    Cl = c_param[:, kk, :, :]
    C = (Cl / MN).transpose(3, 1, 0, 2).reshape(BETA * NT * MN, NT * M1)
    Rrep = jnp.asarray(np.kron(np.eye(M1, dtype=np.float32),
                               np.ones((1, M2), np.float32)))[None]
    W0c = jnp.concatenate([W0[0], W0[1]], axis=1)[None]
    b0c = jnp.concatenate([b0[0], b0[1]], axis=0)[None, None]
    z = jnp.zeros_like(W1[0])
    Wbd = jnp.concatenate([
        jnp.concatenate([W1[0], z], axis=1),
        jnp.concatenate([z, W1[1]], axis=1)], axis=0)[None]
    b1c = jnp.concatenate([b1[0], b1[1]], axis=0)[None, None]
    W2s = jnp.concatenate([W2[0], W2[1]], axis=0)[None]
    b2c = b2.reshape(1, 1, NT)

    grid = N // _NA
    full = lambda *shape: pl.BlockSpec(shape, lambda i: (0,) * len(shape))
    ei, etot = pl.pallas_call(
        _body,
        grid=(grid,),
        in_specs=[
            pl.BlockSpec((_NA, LW), lambda i: (i, 0)),
            pl.BlockSpec((_NA, NT), lambda i: (i, 0)),
            pl.BlockSpec((_NA, 1), lambda i: (i, 0)),
            full(LW, NT * MN),
            full(BETA * NT * MN, NT * M1),
            full(1, M1 * M2, NT * HID),
            full(1, 1, NT * HID),
            full(1, NT * HID, NT * HID),
            full(1, 1, NT * HID),
            full(1, NT * HID, 1),
            full(1, 1, NT),
            full(1, M1, M1 * M2),
        ],
        out_specs=[
            pl.BlockSpec((_NA, 1), lambda i: (i, 0)),
            pl.BlockSpec((1, 1), lambda i: (0, 0)),
        ],
        out_shape=[
            jax.ShapeDtypeStruct((N, 1), jnp.float32),
            jax.ShapeDtypeStruct((1, 1), jnp.float32),
        ],
    )(dr, nn, tm, Sel, C, W0c, b0c, Wbd, b1c, W2s, b2c, Rrep)
    return etot.reshape(B, 1), ei.reshape(B, N)


# speedup vs baseline: 3.4237x; 1.4820x over previous
import functools

import jax
import jax.numpy as jnp
import numpy as np
from jax.experimental import pallas as pl

RC = 6.0
RMIN = 0.5
BETA = 12
M1 = 8
M2 = 8
HID = 64
NT = 2
MN = 64
LW = NT * MN * 4

_NA = 400


def _body(dr_ref, nn_ref, tm_ref, C_ref, W0c_ref, b0c_ref,
          Wbd_ref, b1c_ref, W2s_ref, b2c_ref, Rrep_ref, ei_ref, etot_ref):
    mm = functools.partial(
        jax.lax.dot_general,
        dimension_numbers=(((1,), (0,)), ((), ())),
        precision=jax.lax.Precision.HIGHEST,
        preferred_element_type=jnp.float32,
    )
    r = dr_ref[...] * RC
    na = r.shape[0]
    lane = jax.lax.broadcasted_iota(jnp.int32, (na, NT * MN), 1)
    j = lane & (MN - 1)
    k = lane >> 6
    nnk = jnp.where(k == 0, nn_ref[:, 0:1], nn_ref[:, 1:2])
    s = jnp.clip((r - RMIN) / (RC - RMIN), 0.0, 1.0)
    fc = 1.0 - 3.0 * s * s + 2.0 * s * s * s
    fc = jnp.where(j < nnk, fc, 0.0)
    x = jnp.clip(2.0 * r / RC - 1.0, -1.0, 1.0)

    x2 = 2.0 * x
    us = [fc, fc * x]
    for a in range(2, BETA):
        us.append(x2 * us[-1] - us[-2])
    g = mm(jnp.concatenate(us, axis=1), C_ref[...])

    is0 = tm_ref[...] == 0
    gsel = jnp.where(is0, g[:, :M1], g[:, M1:])
    tile = jnp.concatenate([gsel] * M2, axis=1)
    rep = mm(gsel, Rrep_ref[0])
    feat = rep * tile

    h = jnp.tanh(mm(feat, W0c_ref[0]) + b0c_ref[0])
    h = jnp.tanh(mm(h, Wbd_ref[0]) + b1c_ref[0])
    hlane = jax.lax.broadcasted_iota(jnp.int32, (na, NT * HID), 1)
    tmask = (hlane < HID) == is0
    ei = mm(jnp.where(tmask, h, 0.0), W2s_ref[0])
    ei = ei + jnp.where(is0, b2c_ref[0, 0, 0], b2c_ref[0, 0, 1])
    ei_ref[...] = ei

    prev = jnp.where(pl.program_id(0) == 0, 0.0, etot_ref[0, 0])
    etot_ref[...] = (prev + jnp.sum(ei))[None, None]


def kernel(list_neigh, Imagetype_map, atom_type, ImageDR, num_neigh, nghost,
           c_param, W0, b0, W1, b1, W2, b2):
    B, N, _, _ = list_neigh.shape
    dr = ImageDR.reshape(N, NT * MN, 4)[:, :, 0]
    nn = num_neigh.reshape(N, NT).astype(jnp.int32)
    tm = Imagetype_map.reshape(N, 1).astype(jnp.int32)

    kk = jnp.asarray(np.arange(NT * MN) // MN)
    Cl = c_param[:, kk, :, :]
    C = (Cl / MN).transpose(3, 1, 0, 2).reshape(BETA * NT * MN, NT * M1)
    Rrep = jnp.asarray(np.kron(np.eye(M1, dtype=np.float32),
                               np.ones((1, M2), np.float32)))[None]
    W0c = jnp.concatenate([W0[0], W0[1]], axis=1)[None]
    b0c = jnp.concatenate([b0[0], b0[1]], axis=0)[None, None]
    z = jnp.zeros_like(W1[0])
    Wbd = jnp.concatenate([
        jnp.concatenate([W1[0], z], axis=1),
        jnp.concatenate([z, W1[1]], axis=1)], axis=0)[None]
    b1c = jnp.concatenate([b1[0], b1[1]], axis=0)[None, None]
    W2s = jnp.concatenate([W2[0], W2[1]], axis=0)[None]
    b2c = b2.reshape(1, 1, NT)

    grid = N // _NA
    full = lambda *shape: pl.BlockSpec(shape, lambda i: (0,) * len(shape))
    ei, etot = pl.pallas_call(
        _body,
        grid=(grid,),
        in_specs=[
            pl.BlockSpec((_NA, NT * MN), lambda i: (i, 0)),
            pl.BlockSpec((_NA, NT), lambda i: (i, 0)),
            pl.BlockSpec((_NA, 1), lambda i: (i, 0)),
            full(BETA * NT * MN, NT * M1),
            full(1, M1 * M2, NT * HID),
            full(1, 1, NT * HID),
            full(1, NT * HID, NT * HID),
            full(1, 1, NT * HID),
            full(1, NT * HID, 1),
            full(1, 1, NT),
            full(1, M1, M1 * M2),
        ],
        out_specs=[
            pl.BlockSpec((_NA, 1), lambda i: (i, 0)),
            pl.BlockSpec((1, 1), lambda i: (0, 0)),
        ],
        out_shape=[
            jax.ShapeDtypeStruct((N, 1), jnp.float32),
            jax.ShapeDtypeStruct((1, 1), jnp.float32),
        ],
    )(dr, nn, tm, C, W0c, b0c, Wbd, b1c, W2s, b2c, Rrep)
    return etot.reshape(B, 1), ei.reshape(B, N)
